# Initial kernel scaffold; baseline (speedup 1.0000x reference)
import jax
import jax.numpy as jnp
from jax.experimental import pallas as pl


def kernel(x, W_router, W_enc, K_all, V_all):
    raise NotImplementedError("write your pallas kernel here")


# speedup vs baseline: 9.6294x; 9.6294x over previous
import functools
import math

import jax
import jax.numpy as jnp
from jax import lax
from jax.experimental import pallas as pl
from jax.experimental.pallas import tpu as pltpu
from jax.experimental.pallas import tpu_sc as plsc

_B = 64
_D = 1024
_N = 100000
_RANK = 128
_CK = 20
_FK = 10

_BN = 2048
_NB = (_N + _BN - 1) // _BN

_NEG = float("-inf")
_BIG = 2**30


def _coarse_body_v2(x_ref, wr_ref, we_ref, cs_ref, ci_ref, q_ref,
                    a_ref, bs_ref, bi_ref):
    j = pl.program_id(0)
    x = x_ref[...]

    @pl.when(j == 0)
    def _init():
        bs_ref[...] = jnp.full((_B, _CK), _NEG, jnp.float32)
        bi_ref[...] = jnp.full((_B, _CK), _BIG, jnp.int32)
        q_ref[...] = jnp.dot(x, we_ref[...], preferred_element_type=jnp.float32)

    logits = jnp.dot(x, wr_ref[...], preferred_element_type=jnp.float32)
    col0 = j * _BN
    giota = lax.broadcasted_iota(jnp.int32, (_B, _BN), 1) + col0
    a_ref[...] = jnp.where(giota < _N, logits, _NEG)

    pos20 = lax.broadcasted_iota(jnp.int32, (_B, _CK), 1)

    def step(t, flag):
        def active():
            a = a_ref[...]
            m = jnp.max(a, axis=1, keepdims=True)
            hit = a == m
            idx = jnp.min(jnp.where(hit, giota, _BIG), axis=1, keepdims=True)
            a_ref[...] = jnp.where(giota == idx, _NEG, a)
            bs = bs_ref[...]
            bi = bi_ref[...]
            rmin = jnp.min(bs, axis=1, keepdims=True)
            ismin = bs == rmin
            hi = jnp.max(jnp.where(ismin, bi, -1), axis=1, keepdims=True)
            p = jnp.min(jnp.where(ismin & (bi == hi), pos20, _BIG),
                        axis=1, keepdims=True)
            do = (m > rmin) | ((m == rmin) & (idx < hi))
            repl = (pos20 == p) & do
            bs = jnp.where(repl, m, bs)
            bi = jnp.where(repl, idx, bi)
            bs_ref[...] = bs
            bi_ref[...] = bi
            rmin2 = jnp.min(bs, axis=1, keepdims=True)
            return jnp.any(m >= rmin2).astype(jnp.int32)

        return lax.cond(flag != 0, active, lambda: jnp.zeros((), jnp.int32))

    lax.fori_loop(0, _CK, step, jnp.ones((), jnp.int32), unroll=False)

    @pl.when(j == _NB - 1)
    def _emit():
        bs = bs_ref[...]
        bi = bi_ref[...]
        outs = []
        outi = []
        for _ in range(_CK):
            m = jnp.max(bs, axis=1, keepdims=True)
            ism = bs == m
            lo = jnp.min(jnp.where(ism, bi, _BIG), axis=1, keepdims=True)
            sel = ism & (bi == lo)
            p = jnp.min(jnp.where(sel, pos20, _BIG), axis=1, keepdims=True)
            outs.append(m)
            outi.append(lo)
            bs = jnp.where(pos20 == p, _NEG, bs)
        cs_ref[...] = jnp.concatenate(outs, axis=1)
        ci_ref[...] = jnp.concatenate(outi, axis=1)


def _coarse(x2d, w_router, w_enc):
    return pl.pallas_call(
        _coarse_body_v2,
        grid=(_NB,),
        in_specs=[
            pl.BlockSpec((_B, _D), lambda j: (0, 0)),
            pl.BlockSpec((_D, _BN), lambda j: (0, j)),
            pl.BlockSpec((_D, _RANK), lambda j: (0, 0)),
        ],
        out_specs=[
            pl.BlockSpec((_B, _CK), lambda j: (0, 0)),
            pl.BlockSpec((_B, _CK), lambda j: (0, 0)),
            pl.BlockSpec((_B, _RANK), lambda j: (0, 0)),
        ],
        out_shape=[
            jax.ShapeDtypeStruct((_B, _CK), jnp.float32),
            jax.ShapeDtypeStruct((_B, _CK), jnp.int32),
            jax.ShapeDtypeStruct((_B, _RANK), jnp.float32),
        ],
        scratch_shapes=[
            pltpu.VMEM((_B, _BN), jnp.float32),
            pltpu.VMEM((_B, _CK), jnp.float32),
            pltpu.VMEM((_B, _CK), jnp.int32),
        ],
        compiler_params=pltpu.CompilerParams(
            dimension_semantics=("arbitrary",),
        ),
    )(x2d, w_router, w_enc)


_CIP = 32
_INV_SQRT_RANK = 1.0 / math.sqrt(_RANK)


def _bf16_rne(x):
    u = plsc.bitcast(x, jnp.uint32)
    odd = jax.lax.shift_right_logical(u, jnp.uint32(16)) & jnp.uint32(1)
    r = (u + jnp.uint32(0x7FFF) + odd) & jnp.uint32(0xFFFF0000)
    return plsc.bitcast(r, jnp.float32)


def _fine_body(q_hbm, ci_hbm, k_hbm, v_hbm, out_hbm, fw_hbm, fgi_hbm,
               q_v, ci_v, krows_v, vidx_v, vrows_v, orow_v, fw_v, sem):
    info = plsc.get_sparse_core_info()
    nc = info.num_cores
    wid = lax.axis_index("s") * nc + lax.axis_index("c")
    lanes = lax.broadcasted_iota(jnp.int32, (16,), 0)

    def one_row(r, _):
        b = wid * 2 + r
        pltpu.sync_copy(q_hbm.at[b], q_v)
        pltpu.sync_copy(ci_hbm.at[b], ci_v)
        pltpu.async_copy(k_hbm.at[ci_v], krows_v, sem).wait()

        s0 = jnp.full((16,), _NEG, jnp.float32)
        s1 = jnp.full((16,), _NEG, jnp.float32)
        qb = [_bf16_rne(q_v[pl.ds(h * 16, 16)]) for h in range(_RANK // 16)]
        for c in range(_CK):
            acc = jnp.zeros((16,), jnp.float32)
            for h in range(_RANK // 16):
                acc = acc + (_bf16_rne(krows_v[c, pl.ds(h * 16, 16)])
                             * qb[h])
            sc = jax.lax.reduce_sum_p.bind(acc, axes=(0,)) * _INV_SQRT_RANK
            scv = jnp.full((16,), sc, jnp.float32)
            if c < 16:
                s0 = jnp.where(lanes == c, scv, s0)
            else:
                s1 = jnp.where(lanes == (c - 16), scv, s1)

        i0 = ci_v[pl.ds(0, 16)]
        i1 = ci_v[pl.ds(16, 16)]

        big = jnp.int32(99)
        topw = jnp.full((16,), _NEG, jnp.float32)
        topi = jnp.zeros((16,), jnp.int32)
        for t in range(_FK):
            mm = jnp.maximum(s0, s1)
            m = jax.lax.reduce_max_p.bind(mm, axes=(0,))
            e0 = s0 == m
            e1 = s1 == m
            p0 = jax.lax.reduce_min_p.bind(
                jnp.where(e0, lanes, big), axes=(0,))
            p1 = jax.lax.reduce_min_p.bind(
                jnp.where(e1, lanes + 16, big), axes=(0,))
            p = jnp.minimum(p0, p1)
            gi = (jax.lax.reduce_sum_p.bind(
                      jnp.where(lanes == p, i0, 0), axes=(0,))
                  + jax.lax.reduce_sum_p.bind(
                      jnp.where(lanes + 16 == p, i1, 0), axes=(0,)))
            topw = jnp.where(lanes == t, jnp.full((16,), m, jnp.float32), topw)
            topi = jnp.where(lanes == t, jnp.full((16,), gi, jnp.int32), topi)
            s0 = jnp.where(lanes == p, _NEG, s0)
            s1 = jnp.where(lanes + 16 == p, _NEG, s1)

        mx = jax.lax.reduce_max_p.bind(topw, axes=(0,))
        ex = jnp.exp(topw - mx)
        den = jax.lax.reduce_sum_p.bind(ex, axes=(0,))
        wts = ex / den
        fw_v[...] = wts
        vidx_v[...] = topi
        pltpu.sync_copy(fw_v, fw_hbm.at[b])
        pltpu.sync_copy(vidx_v, fgi_hbm.at[b])

        pltpu.async_copy(v_hbm.at[vidx_v], vrows_v, sem).wait()

        wsc = []
        for t in range(_FK):
            wsc.append(jnp.full((16,), jax.lax.reduce_sum_p.bind(
                jnp.where(lanes == t, wts, jnp.float32(0)), axes=(0,)),
                jnp.float32))
        for h in range(_D // 16):
            acc = jnp.zeros((16,), jnp.float32)
            for t in range(_FK):
                acc = acc + vrows_v[t, pl.ds(h * 16, 16)] * wsc[t]
            orow_v[pl.ds(h * 16, 16)] = acc
        pltpu.sync_copy(orow_v, out_hbm.at[b])
        return ()

    lax.fori_loop(0, 2, one_row, ())


def _fine(q, ci_pad, k_all, v_all):
    mesh = plsc.VectorSubcoreMesh(core_axis_name="c", subcore_axis_name="s")
    f = pl.kernel(
        _fine_body,
        out_type=[
            jax.ShapeDtypeStruct((_B, _D), jnp.float32),
            jax.ShapeDtypeStruct((_B, 16), jnp.float32),
            jax.ShapeDtypeStruct((_B, 16), jnp.int32),
        ],
        mesh=mesh,
        scratch_types=[
            pltpu.VMEM((_RANK,), jnp.float32),
            pltpu.VMEM((_CIP,), jnp.int32),
            pltpu.VMEM((_CIP, _RANK), jnp.float32),
            pltpu.VMEM((16,), jnp.int32),
            pltpu.VMEM((16, _D), jnp.float32),
            pltpu.VMEM((_D,), jnp.float32),
            pltpu.VMEM((16,), jnp.float32),
            pltpu.SemaphoreType.DMA,
        ],
        compiler_params=pltpu.CompilerParams(needs_layout_passes=False),
    )
    return f(q, ci_pad, k_all, v_all)


@jax.jit
def kernel(x, W_router, W_enc, K_all, V_all):
    x2d = x.reshape(_B, _D)
    coarse_scores, candidate_idx, query = _coarse(x2d, W_router, W_enc)
    ci_pad = jnp.pad(candidate_idx, ((0, 0), (0, _CIP - _CK)))
    out, fw, fgi = _fine(query, ci_pad, K_all, V_all)
    return (
        out.reshape(_B, 1, _D),
        coarse_scores.reshape(_B, 1, _CK),
        candidate_idx.reshape(_B, 1, _CK),
        fw[:, :_FK].reshape(_B, 1, _FK),
        fgi[:, :_FK].reshape(_B, 1, _FK),
    )
